# baseline (speedup 1.0000x reference)
import functools

import jax
import jax.numpy as jnp
from jax import lax
from jax.experimental import pallas as pl
from jax.experimental.pallas import tpu as pltpu
from jax.experimental.pallas import tpu_sc as plsc

DIM = 1024
HALF = DIM // 2
COND = DIM * 4
BATCH = 4096
NUM_CLASSES = 1000
LOG_MAX_PERIOD = 9.210340371976184


_SC_INFO = plsc.get_sparse_core_info()
_NW = _SC_INFO.num_cores * _SC_INFO.num_subcores
_B_PER_W = BATCH // _NW
_CHUNK = 16
_NCHUNK = _B_PER_W // _CHUNK


@functools.partial(
    pl.kernel,
    mesh=plsc.VectorSubcoreMesh(core_axis_name="c", subcore_axis_name="s"),
    out_type=jax.ShapeDtypeStruct((BATCH, COND), jnp.float32),
    scratch_types=[
        pltpu.VMEM((_NCHUNK, _CHUNK), jnp.int32),
        pltpu.VMEM((_CHUNK, COND), jnp.float32),
        pltpu.SemaphoreType.DMA,
    ],
)
def _sc_gather(table_hbm, idx_hbm, out_hbm, idx_v, rows_v, sem):
    wid = lax.axis_index("s") * _SC_INFO.num_cores + lax.axis_index("c")
    base = wid * _B_PER_W
    pltpu.sync_copy(idx_hbm.at[wid], idx_v)
    for c in range(_NCHUNK):
        pltpu.async_copy(table_hbm.at[idx_v.at[c]], rows_v, sem).wait()
        pltpu.sync_copy(rows_v, out_hbm.at[pl.ds(base + c * _CHUNK, _CHUNK)])


_BB1 = 1024
_NB1 = BATCH // _BB1


_INV_2PI = 0.15915494309189535
_PI2_HI = 6.28125
_PI2_LO = 0.0019353071795864769
_S = (9.99984587e-01, -1.66632582e-01, 8.31238293e-03, -1.93161822e-04,
      2.17321007e-06)
_C = (9.99999443e-01, -4.99995580e-01, 4.16610316e-02, -1.38627433e-03,
      2.42531378e-05, -2.21936942e-07)


def _tc1_body(t_ref, w1_ref, b1_ref, h_ref):
    t = t_ref[...]
    half_iota = lax.broadcasted_iota(jnp.int32, (1, HALF), 1).astype(jnp.float32)
    freqs = jnp.exp(half_iota * (-LOG_MAX_PERIOD / HALF))
    args = t * freqs
    q = jnp.floor(args * _INV_2PI + 0.5)
    r = (args - q * _PI2_HI) - q * _PI2_LO
    r2 = r * r
    sinr = r * (_S[0] + r2 * (_S[1] + r2 * (_S[2] + r2 * (_S[3] + r2 * _S[4]))))
    cosr = _C[0] + r2 * (_C[1] + r2 * (_C[2] + r2 * (_C[3] + r2 * (_C[4] + r2 * _C[5]))))
    emb = jnp.concatenate([cosr, sinr], axis=1)
    h = jnp.dot(
        emb.astype(jnp.bfloat16),
        w1_ref[...].astype(jnp.bfloat16),
        preferred_element_type=jnp.float32,
    )
    h = h + b1_ref[...]
    h_ref[...] = (h * jax.nn.sigmoid(h)).astype(jnp.bfloat16)


def _tc_stage1(t2d, W1, b1):
    return pl.pallas_call(
        _tc1_body,
        grid=(_NB1,),
        in_specs=[
            pl.BlockSpec((_BB1, 1), lambda i: (i, 0)),
            pl.BlockSpec((DIM, COND), lambda i: (0, 0)),
            pl.BlockSpec((1, COND), lambda i: (0, 0)),
        ],
        out_specs=pl.BlockSpec((_BB1, COND), lambda i: (i, 0)),
        out_shape=jax.ShapeDtypeStruct((BATCH, COND), jnp.bfloat16),
        compiler_params=pltpu.CompilerParams(
            vmem_limit_bytes=100 * 1024 * 1024,
        ),
    )(t2d, W1, b1)


_BB2 = 2048
_NB2 = BATCH // _BB2
_CB = 512
_NC = COND // _CB


def _tc2_body(h_ref, w2_ref, b2_ref, c_ref, out_ref):
    acc = jnp.dot(
        h_ref[...],
        w2_ref[...],
        preferred_element_type=jnp.float32,
    )
    out_ref[...] = acc + b2_ref[...] + c_ref[...]


def _tc_stage2(h, W2, b2, c_emb):
    return pl.pallas_call(
        _tc2_body,
        grid=(_NB2, _NC),
        in_specs=[
            pl.BlockSpec((_BB2, COND), lambda i, j: (i, 0)),
            pl.BlockSpec((COND, _CB), lambda i, j: (0, j)),
            pl.BlockSpec((1, _CB), lambda i, j: (0, j)),
            pl.BlockSpec((_BB2, _CB), lambda i, j: (i, j)),
        ],
        out_specs=pl.BlockSpec((_BB2, _CB), lambda i, j: (i, j)),
        out_shape=jax.ShapeDtypeStruct((BATCH, COND), jnp.float32),
        compiler_params=pltpu.CompilerParams(
            vmem_limit_bytes=63 * 1024 * 1024,
        ),
    )(h, W2, b2, c_emb)


def kernel(timestep, class_label, W1, b1, W2, b2, class_table):
    c_emb = _sc_gather(
        class_table,
        class_label.astype(jnp.int32).reshape(_NW, _NCHUNK, _CHUNK),
    )
    h = _tc_stage1(
        timestep.reshape(BATCH, 1),
        W1,
        b1.reshape(1, COND),
    )
    return _tc_stage2(h, W2.astype(jnp.bfloat16), b2.reshape(1, COND), c_emb)


# speedup vs baseline: 1.2317x; 1.2317x over previous
import functools

import jax
import jax.numpy as jnp
from jax import lax
from jax.experimental import pallas as pl
from jax.experimental.pallas import tpu as pltpu
from jax.experimental.pallas import tpu_sc as plsc

DIM = 1024
HALF = DIM // 2
COND = DIM * 4
BATCH = 4096
NUM_CLASSES = 1000
LOG_MAX_PERIOD = 9.210340371976184


_SC_INFO = plsc.get_sparse_core_info()
_NW = _SC_INFO.num_cores * _SC_INFO.num_subcores
_B_PER_W = BATCH // _NW
_CHUNK = 8
_NCHUNK = _B_PER_W // _CHUNK


@functools.partial(
    pl.kernel,
    mesh=plsc.VectorSubcoreMesh(core_axis_name="c", subcore_axis_name="s"),
    out_type=jax.ShapeDtypeStruct((BATCH, COND), jnp.float32),
    scratch_types=[
        pltpu.VMEM((_NCHUNK, _CHUNK), jnp.int32),
        pltpu.VMEM((_CHUNK, COND), jnp.float32),
        pltpu.VMEM((_CHUNK, COND), jnp.float32),
        pltpu.SemaphoreType.DMA,
        pltpu.SemaphoreType.DMA,
        pltpu.SemaphoreType.DMA,
        pltpu.SemaphoreType.DMA,
    ],
)
def _sc_gather(table_hbm, idx_hbm, out_hbm, idx_v, rows0, rows1,
               gsem0, gsem1, ssem0, ssem1):
    wid = lax.axis_index("s") * _SC_INFO.num_cores + lax.axis_index("c")
    base = wid * _B_PER_W
    rows = (rows0, rows1)
    gsem = (gsem0, gsem1)
    ssem = (ssem0, ssem1)
    pltpu.sync_copy(idx_hbm.at[wid], idx_v)
    gathers = [None] * _NCHUNK
    scatters = [None] * _NCHUNK
    gathers[0] = pltpu.async_copy(table_hbm.at[idx_v.at[0]], rows[0], gsem[0])
    for c in range(_NCHUNK):
        b = c & 1
        gathers[c].wait()
        scatters[c] = pltpu.async_copy(
            rows[b], out_hbm.at[pl.ds(base + c * _CHUNK, _CHUNK)], ssem[b]
        )
        if c + 1 < _NCHUNK:
            if c >= 1:
                scatters[c - 1].wait()
            gathers[c + 1] = pltpu.async_copy(
                table_hbm.at[idx_v.at[c + 1]], rows[1 - b], gsem[1 - b]
            )
    scatters[_NCHUNK - 2].wait()
    scatters[_NCHUNK - 1].wait()


_BB1 = 1024
_NB1 = BATCH // _BB1


_INV_2PI = 0.15915494309189535
_PI2_HI = 6.28125
_PI2_LO = 0.0019353071795864769
_S = (9.99984587e-01, -1.66632582e-01, 8.31238293e-03, -1.93161822e-04,
      2.17321007e-06)
_C = (9.99999443e-01, -4.99995580e-01, 4.16610316e-02, -1.38627433e-03,
      2.42531378e-05, -2.21936942e-07)


def _tc1_body(t_ref, w1_ref, b1_ref, h_ref):
    t = t_ref[...]
    half_iota = lax.broadcasted_iota(jnp.int32, (1, HALF), 1).astype(jnp.float32)
    freqs = jnp.exp(half_iota * (-LOG_MAX_PERIOD / HALF))
    args = t * freqs
    q = jnp.floor(args * _INV_2PI + 0.5)
    r = (args - q * _PI2_HI) - q * _PI2_LO
    r2 = r * r
    sinr = r * (_S[0] + r2 * (_S[1] + r2 * (_S[2] + r2 * (_S[3] + r2 * _S[4]))))
    cosr = _C[0] + r2 * (_C[1] + r2 * (_C[2] + r2 * (_C[3] + r2 * (_C[4] + r2 * _C[5]))))
    emb = jnp.concatenate([cosr, sinr], axis=1)
    h = jnp.dot(
        emb.astype(jnp.bfloat16),
        w1_ref[...].astype(jnp.bfloat16),
        preferred_element_type=jnp.float32,
    )
    h = h + b1_ref[...]
    h_ref[...] = (h * jax.nn.sigmoid(h)).astype(jnp.bfloat16)


def _tc_stage1(t2d, W1, b1):
    return pl.pallas_call(
        _tc1_body,
        grid=(_NB1,),
        in_specs=[
            pl.BlockSpec((_BB1, 1), lambda i: (i, 0)),
            pl.BlockSpec((DIM, COND), lambda i: (0, 0)),
            pl.BlockSpec((1, COND), lambda i: (0, 0)),
        ],
        out_specs=pl.BlockSpec((_BB1, COND), lambda i: (i, 0)),
        out_shape=jax.ShapeDtypeStruct((BATCH, COND), jnp.bfloat16),
        compiler_params=pltpu.CompilerParams(
            vmem_limit_bytes=100 * 1024 * 1024,
        ),
    )(t2d, W1, b1)


_BB2 = 2048
_NB2 = BATCH // _BB2
_CB = 256
_NC = COND // _CB


def _tc2_body(h_ref, w2_ref, b2_ref, c_ref, out_ref):
    acc = jnp.dot(
        h_ref[...],
        w2_ref[...].astype(jnp.bfloat16),
        preferred_element_type=jnp.float32,
    )
    out_ref[...] = acc + b2_ref[...] + c_ref[...]


def _tc_stage2(h, W2, b2, c_emb):
    return pl.pallas_call(
        _tc2_body,
        grid=(_NB2, _NC),
        in_specs=[
            pl.BlockSpec((_BB2, COND), lambda i, j: (i, 0)),
            pl.BlockSpec((COND, _CB), lambda i, j: (0, j)),
            pl.BlockSpec((1, _CB), lambda i, j: (0, j)),
            pl.BlockSpec((_BB2, _CB), lambda i, j: (i, j)),
        ],
        out_specs=pl.BlockSpec((_BB2, _CB), lambda i, j: (i, j)),
        out_shape=jax.ShapeDtypeStruct((BATCH, COND), jnp.float32),
        compiler_params=pltpu.CompilerParams(
            vmem_limit_bytes=63 * 1024 * 1024,
        ),
    )(h, W2, b2, c_emb)


def kernel(timestep, class_label, W1, b1, W2, b2, class_table):
    c_emb = _sc_gather(
        class_table,
        class_label.astype(jnp.int32).reshape(_NW, _NCHUNK, _CHUNK),
    )
    h = _tc_stage1(
        timestep.reshape(BATCH, 1),
        W1,
        b1.reshape(1, COND),
    )
    return _tc_stage2(h, W2, b2.reshape(1, COND), c_emb)
